# baseline (speedup 1.0000x reference)
import jax
import jax.numpy as jnp
from jax import lax
from jax.experimental import pallas as pl
from jax.experimental.pallas import tpu as pltpu
from jax.experimental.pallas import tpu_sc as plsc

_B = 128
_N = 32768
_K = 256
_L = 16
_NC = 2
_NS = 16
_NW = _NC * _NS
_RPW = _B // _NW
_NV = _N // _L
_H = 4096
_HV = _H // _L
_HM = 1024
_HMV = _HM // _L
_CAND = 8192

_U32 = jnp.uint32
_I32 = jnp.int32


def _sortable(xv):
    u = lax.bitcast_convert_type(xv, _U32)
    flip = jnp.where(u >= _U32(0x80000000), _U32(0xFFFFFFFF), _U32(0x80000000))
    return u ^ flip


def _unsortable_vec(us_vec):
    bits = jnp.where(us_vec >= _U32(0x80000000), us_vec ^ _U32(0x80000000), ~us_vec)
    return lax.bitcast_convert_type(bits, jnp.float32)


def _resolve_lane(v, cab, t):
    lane = lax.iota(_I32, _L)
    ps = plsc.cumsum(v)
    tot = ps[_L - 1]
    cnt_gt = cab + tot - ps
    cnt_ge = cnt_gt + v
    pred = (cnt_ge >= t) & (cnt_gt < t)
    ln = jnp.sum(jnp.where(pred, lane, 0))
    cgt = jnp.sum(jnp.where(pred, cnt_gt, 0))
    cge = jnp.sum(jnp.where(pred, cnt_ge, 0))
    return ln, cgt, cge


def _scan_desc(hist_ref, psums_ref, nb, t):
    nvec = nb // _L
    ngrp = nvec // _L
    lane = lax.iota(_I32, _L)

    @plsc.parallel_loop(0, nvec, unroll=8)
    def phase_a(i):
        psums_ref[pl.ds(i * _L, _L)] = plsc.cumsum(hist_ref[pl.ds(i * _L, _L)])

    def phase_b(k, carry):
        above, ivec, cab = carry
        g = ngrp - 1 - k
        idx = (g * _L + lane) * _L + (_L - 1)
        tv = plsc.load_gather(psums_ref, [idx])
        cs = plsc.cumsum(tv)
        tot = cs[_L - 1]
        cge_vec = above + (tot - (cs - tv))
        pred = (cge_vec >= t) & (cge_vec - tv < t)
        ivec = ivec + jnp.sum(jnp.where(pred, g * _L + lane, 0))
        cab = cab + jnp.sum(jnp.where(pred, cge_vec - tv, 0))
        return above + tot, ivec, cab

    _, ivec, cab = lax.fori_loop(0, ngrp, phase_b, (_I32(0), _I32(0), _I32(0)))

    ln, cgt, cge = _resolve_lane(hist_ref[pl.ds(ivec * _L, _L)], cab, t)
    return ivec * _L + ln, cgt, cge


def _popcnt(mask):
    return plsc.all_reduce_population_count(mask)[0]


def _body(x_hbm, out_hbm, row0_ref, row1_ref, hist_ref, psums_ref, coarse_ref,
          cidx_ref, sem_in0, sem_in1, sem_out0, sem_out1):
    wid = lax.axis_index("c") * _NS + lax.axis_index("s")
    zeros16 = jnp.zeros((_L,), _I32)
    ones16 = jnp.ones((_L,), _I32)
    lane = lax.iota(_I32, _L)
    rows = (row0_ref, row1_ref)
    sems_in = (sem_in0, sem_in1)
    sems_out = (sem_out0, sem_out1)

    def clear_hist(nvec):
        @plsc.parallel_loop(0, nvec, unroll=8)
        def clr(i):
            hist_ref[pl.ds(i * _L, _L)] = zeros16

    def select_and_mask(row_ref):
        clear_hist(_HV)

        @plsc.parallel_loop(0, _L, unroll=8)
        def clrc(i):
            coarse_ref[pl.ds(i * _L, _L)] = zeros16

        @plsc.parallel_loop(0, _NV, unroll=8)
        def h1(j):
            us = _sortable(row_ref[pl.ds(j * _L, _L)])
            b = (us >> _U32(20)).astype(_I32)
            plsc.addupdate_scatter(hist_ref, [b], ones16)
            plsc.addupdate_scatter(coarse_ref, [b >> 4], ones16)

        cb, ccgt, _ = _scan_desc(coarse_ref, psums_ref, _L * _L, _I32(_K))
        fl, c1, g1 = _resolve_lane(hist_ref[pl.ds(cb * _L, _L)], ccgt, _I32(_K))
        b1 = cb * _L + fl
        m1 = _K - c1
        t1 = g1 - c1
        b1u = b1.astype(_U32)
        small = t1 <= _CAND

        blo = b1u << _U32(20)

        @pl.when(small)
        def _out_and_collect():
            @plsc.parallel_loop(0, _NV, unroll=8, carry=_I32(0))
            def outp(j, off):
                xv = row_ref[pl.ds(j * _L, _L)]
                us = _sortable(xv)
                keep = us >= blo
                row_ref[pl.ds(j * _L, _L)] = jnp.where(keep, xv, 0.0)
                sel = (us >> _U32(20)) == b1u
                plsc.store_compressed(cidx_ref.at[pl.ds(off, _L)],
                                      j * _L + lane, mask=sel)
                return off + _popcnt(sel)

            ncv = (t1 + _L - 1) // _L

            def cand_us(j):
                idxv = cidx_ref[pl.ds(j * _L, _L)]
                valid = (j * _L + lane) < t1
                vals = plsc.load_gather(row_ref, [idxv], mask=valid)
                return idxv, valid, vals, _sortable(vals)

            clear_hist(_HMV)

            @plsc.parallel_loop(0, ncv, unroll=4)
            def hA(j):
                _, valid, _, us = cand_us(j)
                b = ((us >> _U32(10)) & _U32(0x3FF)).astype(_I32)
                plsc.addupdate_scatter(hist_ref, [b], ones16, mask=valid)

            bA, cA, _ = _scan_desc(hist_ref, psums_ref, _HM, m1)
            mB = m1 - cA
            pfx22 = (b1u << _U32(10)) | bA.astype(_U32)

            clear_hist(_HMV)

            @plsc.parallel_loop(0, ncv, unroll=4)
            def hB(j):
                _, valid, _, us = cand_us(j)
                sel = ((us >> _U32(10)) == pfx22) & valid
                b = (us & _U32(0x3FF)).astype(_I32)
                plsc.addupdate_scatter(hist_ref, [b], ones16, mask=sel)

            bB, cB, _ = _scan_desc(hist_ref, psums_ref, _HM, mB)
            m3 = mB - cB
            u_star = (pfx22 << _U32(10)) | bB.astype(_U32)

            def decide(j, run):
                idxv, valid, vals, us = cand_us(j)
                tie = (us == u_star) & valid
                rank = run + plsc.cumsum(tie.astype(_I32)) - 1
                keep = ((us > u_star) & valid) | (tie & (rank < m3))
                plsc.store_scatter(row_ref, [idxv],
                                   jnp.where(keep, vals, 0.0), mask=valid)
                return run + _popcnt(tie)

            lax.fori_loop(0, ncv, decide, _I32(0))

        @pl.when(jnp.logical_not(small))
        def _fallback():
            clear_hist(_HMV)

            @plsc.parallel_loop(0, _NV, unroll=8)
            def fA(j):
                us = _sortable(row_ref[pl.ds(j * _L, _L)])
                sel = (us >> _U32(20)) == b1u
                b = ((us >> _U32(10)) & _U32(0x3FF)).astype(_I32)
                plsc.addupdate_scatter(hist_ref, [b], ones16, mask=sel)

            bA, cA, _ = _scan_desc(hist_ref, psums_ref, _HM, m1)
            mB = m1 - cA
            pfx22 = (b1u << _U32(10)) | bA.astype(_U32)

            clear_hist(_HMV)

            @plsc.parallel_loop(0, _NV, unroll=8)
            def fB(j):
                us = _sortable(row_ref[pl.ds(j * _L, _L)])
                sel = (us >> _U32(10)) == pfx22
                b = (us & _U32(0x3FF)).astype(_I32)
                plsc.addupdate_scatter(hist_ref, [b], ones16, mask=sel)

            bB, cB, _ = _scan_desc(hist_ref, psums_ref, _HM, mB)
            m3 = mB - cB
            u_star = (pfx22 << _U32(10)) | bB.astype(_U32)

            def decide(j, run):
                xv = row_ref[pl.ds(j * _L, _L)]
                us = _sortable(xv)
                tie = us == u_star
                rank = run + plsc.cumsum(tie.astype(_I32)) - 1
                keep = (us > u_star) | (tie & (rank < m3))
                row_ref[pl.ds(j * _L, _L)] = jnp.where(keep, xv, 0.0)
                return run + _popcnt(tie)

            lax.fori_loop(0, _NV, decide, _I32(0))

    base = wid * _RPW
    in_cp = [None] * _RPW
    out_cp = [None] * _RPW
    in_cp[0] = pltpu.async_copy(x_hbm.at[base], rows[0], sems_in[0])
    for rr in range(_RPW):
        buf = rows[rr % 2]
        in_cp[rr].wait()
        if rr + 1 < _RPW:
            if rr >= 1:
                out_cp[rr - 1].wait()
            in_cp[rr + 1] = pltpu.async_copy(
                x_hbm.at[base + rr + 1], rows[(rr + 1) % 2], sems_in[(rr + 1) % 2])
        select_and_mask(buf)
        out_cp[rr] = pltpu.async_copy(buf, out_hbm.at[base + rr],
                                      sems_out[rr % 2])
    out_cp[_RPW - 2].wait()
    out_cp[_RPW - 1].wait()


_sparsify = pl.kernel(
    _body,
    out_type=jax.ShapeDtypeStruct((_B, _N), jnp.float32),
    mesh=plsc.VectorSubcoreMesh(core_axis_name="c", subcore_axis_name="s"),
    compiler_params=pltpu.CompilerParams(needs_layout_passes=False),
    scratch_types=[
        pltpu.VMEM((_N,), jnp.float32),
        pltpu.VMEM((_N,), jnp.float32),
        pltpu.VMEM((_H,), _I32),
        pltpu.VMEM((_H,), _I32),
        pltpu.VMEM((_L * _L,), _I32),
        pltpu.VMEM((_CAND + _L,), _I32),
        pltpu.SemaphoreType.DMA,
        pltpu.SemaphoreType.DMA,
        pltpu.SemaphoreType.DMA,
        pltpu.SemaphoreType.DMA,
    ],
)


def kernel(x):
    return _sparsify(x)


# speedup vs baseline: 1.3318x; 1.3318x over previous
import jax
import jax.numpy as jnp
from jax import lax
from jax.experimental import pallas as pl
from jax.experimental.pallas import tpu as pltpu
from jax.experimental.pallas import tpu_sc as plsc

_B = 128
_N = 32768
_K = 256
_L = 16
_NC = 2
_NS = 16
_NW = _NC * _NS
_RPW = _B // _NW
_NV = _N // _L
_H = 4096
_HV = _H // _L
_HM = 1024
_HMV = _HM // _L
_CAND = 8192

_U32 = jnp.uint32
_I32 = jnp.int32


def _sortable(xv):
    u = lax.bitcast_convert_type(xv, _U32)
    flip = jnp.where(u >= _U32(0x80000000), _U32(0xFFFFFFFF), _U32(0x80000000))
    return u ^ flip


def _unsortable_vec(us_vec):
    bits = jnp.where(us_vec >= _U32(0x80000000), us_vec ^ _U32(0x80000000), ~us_vec)
    return lax.bitcast_convert_type(bits, jnp.float32)


def _resolve_lane(v, cab, t):
    lane = lax.iota(_I32, _L)
    ps = plsc.cumsum(v)
    tot = ps[_L - 1]
    cnt_gt = cab + tot - ps
    cnt_ge = cnt_gt + v
    pred = (cnt_ge >= t) & (cnt_gt < t)
    ln = jnp.sum(jnp.where(pred, lane, 0))
    cgt = jnp.sum(jnp.where(pred, cnt_gt, 0))
    cge = jnp.sum(jnp.where(pred, cnt_ge, 0))
    return ln, cgt, cge


def _scan_desc(hist_ref, psums_ref, nb, t):
    nvec = nb // _L
    ngrp = nvec // _L
    lane = lax.iota(_I32, _L)

    @plsc.parallel_loop(0, nvec, unroll=8)
    def phase_a(i):
        psums_ref[pl.ds(i * _L, _L)] = plsc.cumsum(hist_ref[pl.ds(i * _L, _L)])

    def phase_b(k, carry):
        above, ivec, cab = carry
        g = ngrp - 1 - k
        idx = (g * _L + lane) * _L + (_L - 1)
        tv = plsc.load_gather(psums_ref, [idx])
        cs = plsc.cumsum(tv)
        tot = cs[_L - 1]
        cge_vec = above + (tot - (cs - tv))
        pred = (cge_vec >= t) & (cge_vec - tv < t)
        ivec = ivec + jnp.sum(jnp.where(pred, g * _L + lane, 0))
        cab = cab + jnp.sum(jnp.where(pred, cge_vec - tv, 0))
        return above + tot, ivec, cab

    _, ivec, cab = lax.fori_loop(0, ngrp, phase_b, (_I32(0), _I32(0), _I32(0)))

    ln, cgt, cge = _resolve_lane(hist_ref[pl.ds(ivec * _L, _L)], cab, t)
    return ivec * _L + ln, cgt, cge


def _popcnt(mask):
    return plsc.all_reduce_population_count(mask)[0]


def _body(x_hbm, out_hbm, row0_ref, row1_ref, hist_ref, psums_ref,
          cidx_ref, sem_in0, sem_in1, sem_out0, sem_out1):
    wid = lax.axis_index("c") * _NS + lax.axis_index("s")
    zeros16 = jnp.zeros((_L,), _I32)
    ones16 = jnp.ones((_L,), _I32)
    lane = lax.iota(_I32, _L)
    rows = (row0_ref, row1_ref)
    sems_in = (sem_in0, sem_in1)
    sems_out = (sem_out0, sem_out1)

    def clear_hist(nvec):
        @plsc.parallel_loop(0, nvec, unroll=8)
        def clr(i):
            hist_ref[pl.ds(i * _L, _L)] = zeros16

    def select_and_mask(row_ref):
        clear_hist(_HV)

        @plsc.parallel_loop(0, _NV, unroll=8)
        def h1(j):
            us = _sortable(row_ref[pl.ds(j * _L, _L)])
            b = (us >> _U32(20)).astype(_I32)
            plsc.addupdate_scatter(hist_ref, [b], ones16)

        b1, c1, g1 = _scan_desc(hist_ref, psums_ref, _H, _I32(_K))
        m1 = _K - c1
        t1 = g1 - c1
        b1u = b1.astype(_U32)
        small = t1 <= _CAND

        blo = b1u << _U32(20)

        @pl.when(small)
        def _out_and_collect():
            @plsc.parallel_loop(0, _NV, unroll=8, carry=_I32(0))
            def outp(j, off):
                xv = row_ref[pl.ds(j * _L, _L)]
                us = _sortable(xv)
                keep = us >= blo
                row_ref[pl.ds(j * _L, _L)] = jnp.where(keep, xv, 0.0)
                sel = (us >> _U32(20)) == b1u
                plsc.store_compressed(cidx_ref.at[pl.ds(off, _L)],
                                      j * _L + lane, mask=sel)
                return off + _popcnt(sel)

            ncv = (t1 + _L - 1) // _L

            def cand_us(j):
                idxv = cidx_ref[pl.ds(j * _L, _L)]
                valid = (j * _L + lane) < t1
                vals = plsc.load_gather(row_ref, [idxv], mask=valid)
                return idxv, valid, vals, _sortable(vals)

            clear_hist(_HMV)

            @plsc.parallel_loop(0, ncv, unroll=4)
            def hA(j):
                _, valid, _, us = cand_us(j)
                b = ((us >> _U32(10)) & _U32(0x3FF)).astype(_I32)
                plsc.addupdate_scatter(hist_ref, [b], ones16, mask=valid)

            bA, cA, _ = _scan_desc(hist_ref, psums_ref, _HM, m1)
            mB = m1 - cA
            pfx22 = (b1u << _U32(10)) | bA.astype(_U32)

            clear_hist(_HMV)

            @plsc.parallel_loop(0, ncv, unroll=4)
            def hB(j):
                _, valid, _, us = cand_us(j)
                sel = ((us >> _U32(10)) == pfx22) & valid
                b = (us & _U32(0x3FF)).astype(_I32)
                plsc.addupdate_scatter(hist_ref, [b], ones16, mask=sel)

            bB, cB, _ = _scan_desc(hist_ref, psums_ref, _HM, mB)
            m3 = mB - cB
            u_star = (pfx22 << _U32(10)) | bB.astype(_U32)

            def decide(j, run):
                idxv, valid, vals, us = cand_us(j)
                tie = (us == u_star) & valid
                rank = run + plsc.cumsum(tie.astype(_I32)) - 1
                keep = ((us > u_star) & valid) | (tie & (rank < m3))
                plsc.store_scatter(row_ref, [idxv],
                                   jnp.where(keep, vals, 0.0), mask=valid)
                return run + _popcnt(tie)

            lax.fori_loop(0, ncv, decide, _I32(0))

        @pl.when(jnp.logical_not(small))
        def _fallback():
            clear_hist(_HMV)

            @plsc.parallel_loop(0, _NV, unroll=8)
            def fA(j):
                us = _sortable(row_ref[pl.ds(j * _L, _L)])
                sel = (us >> _U32(20)) == b1u
                b = ((us >> _U32(10)) & _U32(0x3FF)).astype(_I32)
                plsc.addupdate_scatter(hist_ref, [b], ones16, mask=sel)

            bA, cA, _ = _scan_desc(hist_ref, psums_ref, _HM, m1)
            mB = m1 - cA
            pfx22 = (b1u << _U32(10)) | bA.astype(_U32)

            clear_hist(_HMV)

            @plsc.parallel_loop(0, _NV, unroll=8)
            def fB(j):
                us = _sortable(row_ref[pl.ds(j * _L, _L)])
                sel = (us >> _U32(10)) == pfx22
                b = (us & _U32(0x3FF)).astype(_I32)
                plsc.addupdate_scatter(hist_ref, [b], ones16, mask=sel)

            bB, cB, _ = _scan_desc(hist_ref, psums_ref, _HM, mB)
            m3 = mB - cB
            u_star = (pfx22 << _U32(10)) | bB.astype(_U32)

            def decide(j, run):
                xv = row_ref[pl.ds(j * _L, _L)]
                us = _sortable(xv)
                tie = us == u_star
                rank = run + plsc.cumsum(tie.astype(_I32)) - 1
                keep = (us > u_star) | (tie & (rank < m3))
                row_ref[pl.ds(j * _L, _L)] = jnp.where(keep, xv, 0.0)
                return run + _popcnt(tie)

            lax.fori_loop(0, _NV, decide, _I32(0))

    base = wid * _RPW
    in_cp = [None] * _RPW
    out_cp = [None] * _RPW
    in_cp[0] = pltpu.async_copy(x_hbm.at[base], rows[0], sems_in[0])
    for rr in range(_RPW):
        buf = rows[rr % 2]
        in_cp[rr].wait()
        if rr + 1 < _RPW:
            if rr >= 1:
                out_cp[rr - 1].wait()
            in_cp[rr + 1] = pltpu.async_copy(
                x_hbm.at[base + rr + 1], rows[(rr + 1) % 2], sems_in[(rr + 1) % 2])
        select_and_mask(buf)
        out_cp[rr] = pltpu.async_copy(buf, out_hbm.at[base + rr],
                                      sems_out[rr % 2])
    out_cp[_RPW - 2].wait()
    out_cp[_RPW - 1].wait()


_sparsify = pl.kernel(
    _body,
    out_type=jax.ShapeDtypeStruct((_B, _N), jnp.float32),
    mesh=plsc.VectorSubcoreMesh(core_axis_name="c", subcore_axis_name="s"),
    compiler_params=pltpu.CompilerParams(needs_layout_passes=False),
    scratch_types=[
        pltpu.VMEM((_N,), jnp.float32),
        pltpu.VMEM((_N,), jnp.float32),
        pltpu.VMEM((_H,), _I32),
        pltpu.VMEM((_H,), _I32),
        pltpu.VMEM((_CAND + _L,), _I32),
        pltpu.SemaphoreType.DMA,
        pltpu.SemaphoreType.DMA,
        pltpu.SemaphoreType.DMA,
        pltpu.SemaphoreType.DMA,
    ],
)


def kernel(x):
    return _sparsify(x)
